# baseline (speedup 1.0000x reference)
import jax
import jax.numpy as jnp
from jax.experimental import pallas as pl
from jax.experimental.pallas import tpu as pltpu


def _pick_tc(T):
    for c in (64, 32, 16, 8, 4, 2, 1):
        if T % c == 0:
            return c
    return 1


def _col_scale(G, Hp):
    lane = jax.lax.broadcasted_iota(jnp.int32, (1, G), 1)
    return jnp.where(lane // Hp == 2, 1.0, 0.5).astype(jnp.float32)


def _cell(th, c, Hp):
    i_g = 0.5 + 0.5 * th[:, 0 * Hp:1 * Hp]
    f_g = 0.5 + 0.5 * th[:, 1 * Hp:2 * Hp]
    g_g = th[:, 2 * Hp:3 * Hp]
    o_g = 0.5 + 0.5 * th[:, 3 * Hp:4 * Hp]
    c_new = f_g * c + i_g * g_g
    h_new = o_g * jnp.tanh(c_new)
    return h_new, c_new


def _make_fused_kernel(Tc, B, Hp, nT):
    G = 4 * Hp
    RB = Tc * B

    def body(tok_ref, emb_ref, w0f_ref, w0b_ref, b0f_ref, b0b_ref,
             whh0f_ref, whh0b_ref,
             w1f0_ref, w1f1_ref, w1b0_ref, w1b1_ref, b1f_ref, b1b_ref,
             whh1f_ref, whh1b_ref, fcwf_ref, fcwb_ref, fcb_ref,
             out_ref,
             x_sc, hfseq_sc, hbseq_sc,
             whh0f_sc, whh0b_sc, whh1f_sc, whh1b_sc,
             w1f_sc, w1b_sc, w0f_sc, w0b_sc):
        def _build_weights():
            bf16 = jnp.bfloat16
            csc = _col_scale(G, Hp)
            whh0f_sc[...] = (whh0f_ref[...] * csc).astype(bf16)
            whh0b_sc[...] = (whh0b_ref[...] * csc).astype(bf16)
            whh1f_sc[...] = (whh1f_ref[...] * csc).astype(bf16)
            whh1b_sc[...] = (whh1b_ref[...] * csc).astype(bf16)
            w1f_sc[:Hp, :] = (w1f0_ref[...] * csc).astype(bf16)
            w1f_sc[Hp:, :] = (w1f1_ref[...] * csc).astype(bf16)
            w1b_sc[:Hp, :] = (w1b0_ref[...] * csc).astype(bf16)
            w1b_sc[Hp:, :] = (w1b1_ref[...] * csc).astype(bf16)
            w0f_sc[...] = (w0f_ref[...] * csc).astype(bf16)
            w0b_sc[...] = (w0b_ref[...] * csc).astype(bf16)

        def _gather():
            for r in range(RB):
                tok = tok_ref[r // B, r % B]
                x_sc[pl.ds(r, 1), :] = emb_ref[pl.ds(tok, 1), :]

        def _layer0():
            csc = _col_scale(G, Hp)
            xf = x_sc[...].astype(jnp.bfloat16)
            xb = xf
            pf = jnp.dot(xf, w0f_sc[...],
                         preferred_element_type=jnp.float32) + b0f_ref[...] * csc
            pb = jnp.dot(xb, w0b_sc[...],
                         preferred_element_type=jnp.float32) + b0b_ref[...] * csc
            whf = whh0f_sc[...]
            whb = whh0b_sc[...]
            z = jnp.zeros((B, Hp), jnp.float32)
            hf, cf, hb, cb = z, z, z, z
            for s in range(Tc):
                gdf = jnp.dot(hf.astype(jnp.bfloat16), whf,
                              preferred_element_type=jnp.float32)
                gdb = jnp.dot(hb.astype(jnp.bfloat16), whb,
                              preferred_element_type=jnp.float32)
                thf = jnp.tanh(gdf + pf[s * B:(s + 1) * B])
                thb = jnp.tanh(gdb + pb[(Tc - 1 - s) * B:(Tc - s) * B])
                hf, cf = _cell(thf, cf, Hp)
                hb, cb = _cell(thb, cb, Hp)
                hfseq_sc[pl.ds(s * B, B), :] = hf.astype(jnp.bfloat16)
                hbseq_sc[pl.ds((Tc - 1 - s) * B, B), :] = (
                    hb.astype(jnp.bfloat16))
        def _layer1():
            csc = _col_scale(G, Hp)
            catf = jnp.concatenate(
                [hfseq_sc[...], hbseq_sc[...]], axis=1)
            catb = catf
            pf = jnp.dot(catf, w1f_sc[...],
                         preferred_element_type=jnp.float32) + b1f_ref[...] * csc
            pb = jnp.dot(catb, w1b_sc[...],
                         preferred_element_type=jnp.float32) + b1b_ref[...] * csc
            whf = whh1f_sc[...]
            whb = whh1b_sc[...]
            z = jnp.zeros((B, Hp), jnp.float32)
            hf, cf, hb, cb = z, z, z, z
            hb_first = None
            for s in range(Tc):
                gdf = jnp.dot(hf.astype(jnp.bfloat16), whf,
                              preferred_element_type=jnp.float32)
                gdb = jnp.dot(hb.astype(jnp.bfloat16), whb,
                              preferred_element_type=jnp.float32)
                thf = jnp.tanh(gdf + pf[s * B:(s + 1) * B])
                thb = jnp.tanh(gdb + pb[(Tc - 1 - s) * B:(Tc - s) * B])
                hf, cf = _cell(thf, cf, Hp)
                hb, cb = _cell(thb, cb, Hp)
                if s == 0:
                    hb_first = hb
            logits = (jnp.dot(hb_first, fcwb_ref[...],
                              preferred_element_type=jnp.float32)
                      + fcb_ref[...]
                      + jnp.dot(hf, fcwf_ref[...],
                                preferred_element_type=jnp.float32))
            m = jnp.max(logits, axis=-1, keepdims=True)
            shifted = logits - m
            lse = jnp.log(jnp.sum(jnp.exp(shifted), axis=-1, keepdims=True))
            out_ref[...] = shifted - lse

        _build_weights()
        _gather()
        _layer0()
        _layer1()

    return body


def kernel(embedding, l0_w_in_f0, l0_w_in_b0, l0_b_f, l0_b_b, l0_whh_f,
           l0_whh_b, l1_w_in_f0, l1_w_in_f1, l1_w_in_b0, l1_w_in_b1, l1_b_f,
           l1_b_b, l1_whh_f, l1_whh_b, fc_wf, fc_wb, fc_b, tokens):
    T, B = tokens.shape
    V, E = embedding.shape
    Hp = l0_whh_f.shape[0]
    G = 4 * Hp
    O = fc_wf.shape[1]
    Tc = _pick_tc(T)
    nT = T // Tc

    const = lambda i, tok: (0, 0)

    out = pl.pallas_call(
        _make_fused_kernel(Tc, B, Hp, nT),
        out_shape=jax.ShapeDtypeStruct((B, O), jnp.float32),
        grid_spec=pltpu.PrefetchScalarGridSpec(
            num_scalar_prefetch=1,
            grid=(1,),
            in_specs=[
                pl.BlockSpec((V, E), const),
                pl.BlockSpec((E, G), const),
                pl.BlockSpec((E, G), const),
                pl.BlockSpec((1, G), const),
                pl.BlockSpec((1, G), const),
                pl.BlockSpec((Hp, G), const),
                pl.BlockSpec((Hp, G), const),
                pl.BlockSpec((Hp, G), const),
                pl.BlockSpec((Hp, G), const),
                pl.BlockSpec((Hp, G), const),
                pl.BlockSpec((Hp, G), const),
                pl.BlockSpec((1, G), const),
                pl.BlockSpec((1, G), const),
                pl.BlockSpec((Hp, G), const),
                pl.BlockSpec((Hp, G), const),
                pl.BlockSpec((Hp, O), const),
                pl.BlockSpec((Hp, O), const),
                pl.BlockSpec((1, O), const),
            ],
            out_specs=pl.BlockSpec((B, O), const),
            scratch_shapes=[
                pltpu.VMEM((T * B, E), jnp.float32),
                pltpu.VMEM((T * B, Hp), jnp.bfloat16),
                pltpu.VMEM((T * B, Hp), jnp.bfloat16),
                pltpu.VMEM((Hp, G), jnp.bfloat16),
                pltpu.VMEM((Hp, G), jnp.bfloat16),
                pltpu.VMEM((Hp, G), jnp.bfloat16),
                pltpu.VMEM((Hp, G), jnp.bfloat16),
                pltpu.VMEM((2 * Hp, G), jnp.bfloat16),
                pltpu.VMEM((2 * Hp, G), jnp.bfloat16),
                pltpu.VMEM((E, G), jnp.bfloat16),
                pltpu.VMEM((E, G), jnp.bfloat16),
            ],
        ),
        compiler_params=pltpu.CompilerParams(
            dimension_semantics=("arbitrary",)),
    )(tokens, embedding, l0_w_in_f0, l0_w_in_b0, l0_b_f, l0_b_b,
      l0_whh_f, l0_whh_b, l1_w_in_f0, l1_w_in_f1, l1_w_in_b0, l1_w_in_b1,
      l1_b_f, l1_b_b, l1_whh_f, l1_whh_b, fc_wf, fc_wb, fc_b)

    return out


# speedup vs baseline: 1.0342x; 1.0342x over previous
import jax
import jax.numpy as jnp
from jax.experimental import pallas as pl
from jax.experimental.pallas import tpu as pltpu


def _pick_tc(T):
    for c in (64, 32, 16, 8, 4, 2, 1):
        if T % c == 0:
            return c
    return 1


def _col_scale(G, Hp):
    lane = jax.lax.broadcasted_iota(jnp.int32, (1, G), 1)
    return jnp.where(lane // Hp == 2, 1.0, 0.5).astype(jnp.float32)


def _cell(th, c, Hp):
    i_g = 0.5 + 0.5 * th[:, 0 * Hp:1 * Hp]
    f_g = 0.5 + 0.5 * th[:, 1 * Hp:2 * Hp]
    g_g = th[:, 2 * Hp:3 * Hp]
    o_g = 0.5 + 0.5 * th[:, 3 * Hp:4 * Hp]
    c_new = f_g * c + i_g * g_g
    h_new = o_g * jnp.tanh(c_new)
    return h_new, c_new


def _make_fused_kernel(Tc, B, Hp, nT):
    G = 4 * Hp
    RB = Tc * B

    def body(tok_ref, emb_ref, w0f_ref, w0b_ref, b0f_ref, b0b_ref,
             whh0f_ref, whh0b_ref,
             w1f0_ref, w1f1_ref, w1b0_ref, w1b1_ref, b1f_ref, b1b_ref,
             whh1f_ref, whh1b_ref, fcwf_ref, fcwb_ref, fcb_ref,
             out_ref,
             x_sc, hfseq_sc, hbseq_sc,
             whh0f_sc, whh0b_sc, whh1f_sc, whh1b_sc,
             w1f_sc, w1b_sc, w0f_sc, w0b_sc):
        def _build_weights():
            bf16 = jnp.bfloat16
            csc = _col_scale(G, Hp)
            whh0f_sc[...] = (whh0f_ref[...] * csc).astype(bf16)
            whh0b_sc[...] = (whh0b_ref[...] * csc).astype(bf16)
            whh1f_sc[...] = (whh1f_ref[...] * csc).astype(bf16)
            whh1b_sc[...] = (whh1b_ref[...] * csc).astype(bf16)
            w1f_sc[:Hp, :] = (w1f0_ref[...] * csc).astype(bf16)
            w1f_sc[Hp:, :] = (w1f1_ref[...] * csc).astype(bf16)
            w1b_sc[:Hp, :] = (w1b0_ref[...] * csc).astype(bf16)
            w1b_sc[Hp:, :] = (w1b1_ref[...] * csc).astype(bf16)
            w0f_sc[...] = (w0f_ref[...] * csc).astype(bf16)
            w0b_sc[...] = (w0b_ref[...] * csc).astype(bf16)

        def _gather():
            for r in range(RB):
                tok = tok_ref[r // B, r % B]
                x_sc[pl.ds(r, 1), :] = emb_ref[pl.ds(tok, 1), :]

        def _layer0():
            csc = _col_scale(G, Hp)
            xf = x_sc[...].astype(jnp.bfloat16)
            CH = 8 if Tc % 8 == 0 else Tc
            nCh = Tc // CH
            CR = CH * B
            wf_st = w0f_sc[...]
            wb_st = w0b_sc[...]
            bfv = b0f_ref[...] * csc
            bbv = b0b_ref[...] * csc
            pfc = [None] * nCh
            pbc = [None] * nCh

            def _projf(c):
                return jnp.dot(xf[c * CR:(c + 1) * CR], wf_st,
                               preferred_element_type=jnp.float32) + bfv

            def _projb(c):
                return jnp.dot(xf[c * CR:(c + 1) * CR], wb_st,
                               preferred_element_type=jnp.float32) + bbv

            pfc[0] = _projf(0)
            pbc[nCh - 1] = _projb(nCh - 1)
            whf = whh0f_sc[...]
            whb = whh0b_sc[...]
            z = jnp.zeros((B, Hp), jnp.float32)
            hf, cf, hb, cb = z, z, z, z
            for s in range(Tc):
                c, u = s // CH, s % CH
                if u == 0 and c + 1 < nCh:
                    pfc[c + 1] = _projf(c + 1)
                    pbc[nCh - 2 - c] = _projb(nCh - 2 - c)
                gdf = jnp.dot(hf.astype(jnp.bfloat16), whf,
                              preferred_element_type=jnp.float32)
                gdb = jnp.dot(hb.astype(jnp.bfloat16), whb,
                              preferred_element_type=jnp.float32)
                thf = jnp.tanh(gdf + pfc[c][u * B:(u + 1) * B])
                thb = jnp.tanh(
                    gdb + pbc[nCh - 1 - c][(CH - 1 - u) * B:(CH - u) * B])
                hf, cf = _cell(thf, cf, Hp)
                hb, cb = _cell(thb, cb, Hp)
                hfseq_sc[pl.ds(s * B, B), :] = hf.astype(jnp.bfloat16)
                hbseq_sc[pl.ds((Tc - 1 - s) * B, B), :] = (
                    hb.astype(jnp.bfloat16))
        def _layer1():
            csc = _col_scale(G, Hp)
            catf = jnp.concatenate(
                [hfseq_sc[...], hbseq_sc[...]], axis=1)
            CH = 8 if Tc % 8 == 0 else Tc
            nCh = Tc // CH
            CR = CH * B
            w1f_st = w1f_sc[...]
            w1b_st = w1b_sc[...]
            b1fv = b1f_ref[...] * csc
            b1bv = b1b_ref[...] * csc
            pfc = [None] * nCh
            pbc = [None] * nCh

            def _projf(c):
                return jnp.dot(catf[c * CR:(c + 1) * CR], w1f_st,
                               preferred_element_type=jnp.float32) + b1fv

            def _projb(c):
                return jnp.dot(catf[c * CR:(c + 1) * CR], w1b_st,
                               preferred_element_type=jnp.float32) + b1bv

            pfc[0] = _projf(0)
            pbc[nCh - 1] = _projb(nCh - 1)
            whf = whh1f_sc[...]
            whb = whh1b_sc[...]
            z = jnp.zeros((B, Hp), jnp.float32)
            hf, cf, hb, cb = z, z, z, z
            hb_first = None
            for s in range(Tc):
                c, u = s // CH, s % CH
                if u == 0 and c + 1 < nCh:
                    pfc[c + 1] = _projf(c + 1)
                    pbc[nCh - 2 - c] = _projb(nCh - 2 - c)
                gdf = jnp.dot(hf.astype(jnp.bfloat16), whf,
                              preferred_element_type=jnp.float32)
                gdb = jnp.dot(hb.astype(jnp.bfloat16), whb,
                              preferred_element_type=jnp.float32)
                thf = jnp.tanh(gdf + pfc[c][u * B:(u + 1) * B])
                thb = jnp.tanh(
                    gdb + pbc[nCh - 1 - c][(CH - 1 - u) * B:(CH - u) * B])
                hf, cf = _cell(thf, cf, Hp)
                hb, cb = _cell(thb, cb, Hp)
                if s == 0:
                    hb_first = hb
            logits = (jnp.dot(hb_first, fcwb_ref[...],
                              preferred_element_type=jnp.float32)
                      + fcb_ref[...]
                      + jnp.dot(hf, fcwf_ref[...],
                                preferred_element_type=jnp.float32))
            m = jnp.max(logits, axis=-1, keepdims=True)
            shifted = logits - m
            lse = jnp.log(jnp.sum(jnp.exp(shifted), axis=-1, keepdims=True))
            out_ref[...] = shifted - lse

        _build_weights()
        _gather()
        _layer0()
        _layer1()

    return body


def kernel(embedding, l0_w_in_f0, l0_w_in_b0, l0_b_f, l0_b_b, l0_whh_f,
           l0_whh_b, l1_w_in_f0, l1_w_in_f1, l1_w_in_b0, l1_w_in_b1, l1_b_f,
           l1_b_b, l1_whh_f, l1_whh_b, fc_wf, fc_wb, fc_b, tokens):
    T, B = tokens.shape
    V, E = embedding.shape
    Hp = l0_whh_f.shape[0]
    G = 4 * Hp
    O = fc_wf.shape[1]
    Tc = _pick_tc(T)
    nT = T // Tc

    const = lambda i, tok: (0, 0)

    out = pl.pallas_call(
        _make_fused_kernel(Tc, B, Hp, nT),
        out_shape=jax.ShapeDtypeStruct((B, O), jnp.float32),
        grid_spec=pltpu.PrefetchScalarGridSpec(
            num_scalar_prefetch=1,
            grid=(1,),
            in_specs=[
                pl.BlockSpec((V, E), const),
                pl.BlockSpec((E, G), const),
                pl.BlockSpec((E, G), const),
                pl.BlockSpec((1, G), const),
                pl.BlockSpec((1, G), const),
                pl.BlockSpec((Hp, G), const),
                pl.BlockSpec((Hp, G), const),
                pl.BlockSpec((Hp, G), const),
                pl.BlockSpec((Hp, G), const),
                pl.BlockSpec((Hp, G), const),
                pl.BlockSpec((Hp, G), const),
                pl.BlockSpec((1, G), const),
                pl.BlockSpec((1, G), const),
                pl.BlockSpec((Hp, G), const),
                pl.BlockSpec((Hp, G), const),
                pl.BlockSpec((Hp, O), const),
                pl.BlockSpec((Hp, O), const),
                pl.BlockSpec((1, O), const),
            ],
            out_specs=pl.BlockSpec((B, O), const),
            scratch_shapes=[
                pltpu.VMEM((T * B, E), jnp.float32),
                pltpu.VMEM((T * B, Hp), jnp.bfloat16),
                pltpu.VMEM((T * B, Hp), jnp.bfloat16),
                pltpu.VMEM((Hp, G), jnp.bfloat16),
                pltpu.VMEM((Hp, G), jnp.bfloat16),
                pltpu.VMEM((Hp, G), jnp.bfloat16),
                pltpu.VMEM((Hp, G), jnp.bfloat16),
                pltpu.VMEM((2 * Hp, G), jnp.bfloat16),
                pltpu.VMEM((2 * Hp, G), jnp.bfloat16),
                pltpu.VMEM((E, G), jnp.bfloat16),
                pltpu.VMEM((E, G), jnp.bfloat16),
            ],
        ),
        compiler_params=pltpu.CompilerParams(
            dimension_semantics=("arbitrary",)),
    )(tokens, embedding, l0_w_in_f0, l0_w_in_b0, l0_b_f, l0_b_b,
      l0_whh_f, l0_whh_b, l1_w_in_f0, l1_w_in_f1, l1_w_in_b0, l1_w_in_b1,
      l1_b_f, l1_b_b, l1_whh_f, l1_whh_b, fc_wf, fc_wb, fc_b)

    return out
